# baseline (speedup 1.0000x reference)
import functools

import jax
import jax.numpy as jnp
from jax import lax
from jax.experimental import pallas as pl
from jax.experimental.pallas import tpu as pltpu
from jax.experimental.pallas import tpu_sc as plsc

N = 10000
E = 320000
D_IN = 128
D_OUT = 64
DH = D_OUT // 2

NP = 10240
NTILES = 16
NSC = 2
CHUNK = 128
NCH = 160
EPT = NCH * CHUNK
EPS = EPT * NTILES
ROWS_PER_TILE = NP // NTILES
NBUF = 8

_MESH = plsc.VectorSubcoreMesh(core_axis_name="c", subcore_axis_name="s")


@functools.partial(
    pl.kernel,
    out_type=jax.ShapeDtypeStruct((NSC * NTILES, NP), jnp.float32),
    mesh=_MESH,
    scratch_types=[
        pltpu.VMEM((EPT,), jnp.int32),
        pltpu.VMEM((NP,), jnp.float32),
    ],
    compiler_params=pltpu.CompilerParams(needs_layout_passes=False),
)
def _deg_kernel(dst_hbm, deg_out, idx_v, deg_v):
    c = lax.axis_index("c").astype(jnp.int32)
    s = lax.axis_index("s").astype(jnp.int32)
    wid = c * jnp.int32(NTILES) + s
    pltpu.sync_copy(dst_hbm.at[wid], idx_v)

    zeros16 = jnp.zeros((16,), jnp.float32)

    @pl.loop(jnp.int32(0), jnp.int32(NP // 16))
    def _zero(i):
        deg_v[pl.ds(pl.multiple_of(i * 16, 16), 16)] = zeros16

    ones16 = jnp.ones((16,), jnp.float32)

    @pl.loop(jnp.int32(0), jnp.int32(EPT // 64))
    def _count(i):
        for j in range(4):
            idx = idx_v[pl.ds(pl.multiple_of(i * 64 + j * 16, 16), 16)]
            plsc.addupdate_scatter(deg_v, [idx], ones16)

    pltpu.sync_copy(deg_v, deg_out.at[wid])


@functools.partial(
    pl.kernel,
    out_type=jax.ShapeDtypeStruct((NSC * 2 * NP, DH), jnp.float32),
    mesh=_MESH,
    scratch_types=[
        pltpu.VMEM((NCH, CHUNK), jnp.int32),
        pltpu.VMEM((NCH, CHUNK), jnp.int32),
        [pltpu.VMEM((CHUNK, DH), jnp.float32) for _ in range(NBUF)],
        pltpu.VMEM_SHARED((NP, DH), jnp.float32),
        pltpu.VMEM_SHARED((NP, DH), jnp.float32),
        [pltpu.SemaphoreType.DMA for _ in range(NBUF)],
        [pltpu.SemaphoreType.DMA for _ in range(NBUF)],
    ],
    compiler_params=pltpu.CompilerParams(use_tc_tiling_on_sc=False),
)
def _scatter_kernel(hn_hbm, src_hbm, dst_hbm, s_out,
                    src_v, dst_v, bufs, s_sh, hn_sh, gsems, ssems):
    c = lax.axis_index("c").astype(jnp.int32)
    s = lax.axis_index("s").astype(jnp.int32)
    wid = c * jnp.int32(NTILES) + s
    tile0 = pl.multiple_of(s * jnp.int32(ROWS_PER_TILE), CHUNK)

    pltpu.sync_copy(src_hbm.at[wid], src_v)
    pltpu.sync_copy(dst_hbm.at[wid], dst_v)

    zeros16 = jnp.zeros((16,), jnp.float32)

    def _gather(ch, b):
        return pltpu.make_async_copy(hn_sh.at[src_v.at[ch]], bufs[b], gsems[b])

    def _scat(ch, b):
        return pltpu.make_async_copy(bufs[b], s_sh.at[dst_v.at[ch]], ssems[b])

    for half in range(2):
        @pl.loop(jnp.int32(0), jnp.int32(CHUNK))
        def _zrow(i):
            for j in range(DH // 16):
                bufs[0][i, pl.ds(j * 16, 16)] = zeros16

        half_base = pl.multiple_of((c * jnp.int32(2) + jnp.int32(half))
                                   * jnp.int32(NP) + tile0, CHUNK)
        pltpu.sync_copy(hn_hbm.at[pl.ds(half_base, ROWS_PER_TILE)],
                        hn_sh.at[pl.ds(tile0, ROWS_PER_TILE)])
        for k in range(ROWS_PER_TILE // CHUNK):
            row0 = pl.multiple_of(tile0 + jnp.int32(k * CHUNK), CHUNK)
            pltpu.sync_copy(bufs[0], s_sh.at[pl.ds(row0, CHUNK)])

        plsc.subcore_barrier()

        for b in range(NBUF):
            _gather(jnp.int32(b), b).start()

        @pl.loop(jnp.int32(0), jnp.int32(NCH), step=jnp.int32(NBUF))
        def _main(g0):
            for b in range(NBUF):
                ch = g0 + b
                _gather(ch, b).wait()
                pltpu.async_copy(bufs[b], s_sh.at[dst_v.at[ch]], ssems[b],
                                 add=True)
                nxt = ch + NBUF

                @pl.when(nxt < NCH)
                def _start_next():
                    _scat(ch, b).wait()
                    _gather(nxt, b).start()

            @pl.when(g0 + jnp.int32(NBUF) >= jnp.int32(NCH))
            def _drain():
                for b in range(NBUF):
                    _scat(g0 + b, b).wait()

        plsc.subcore_barrier()

        pltpu.sync_copy(s_sh.at[pl.ds(tile0, ROWS_PER_TILE)],
                        s_out.at[pl.ds(half_base, ROWS_PER_TILE)])

        if half == 0:
            plsc.subcore_barrier()


def _tc1_body(x_ref, w_ref, deg_ref, hn_ref, dinv_ref):
    h = jnp.dot(x_ref[...], w_ref[...], preferred_element_type=jnp.float32)
    deg = deg_ref[...].reshape(NSC, NTILES, NP).sum(axis=1) + 1.0
    rows = lax.broadcasted_iota(jnp.int32, (NSC, NP), 1)
    dinv = jnp.where(rows < N, lax.rsqrt(deg), 0.0)
    dinv_ref[...] = dinv
    for e in range(NSC):
        hne = h * dinv[e][:, None]
        for q in range(2):
            base = (e * 2 + q) * NP
            hn_ref[base:base + NP, :] = hne[:, q * DH:(q + 1) * DH]


def _tc1(x_pad, w, deg_parts):
    return pl.pallas_call(
        _tc1_body,
        out_shape=(
            jax.ShapeDtypeStruct((NSC * 2 * NP, DH), jnp.float32),
            jax.ShapeDtypeStruct((NSC, NP), jnp.float32),
        ),
    )(x_pad, w, deg_parts)


def _tc2_body(s_ref, hn_ref, dinv_ref, b_ref, o_ref):
    dinv = dinv_ref[...]
    bias = b_ref[...]
    for e in range(NSC):
        for q in range(2):
            base = (e * 2 + q) * NP
            col = e * D_OUT + q * DH
            o_ref[:, col:col + DH] = (
                dinv[e][:, None]
                * (s_ref[base:base + NP, :] + hn_ref[base:base + NP, :])
                + bias[:, q * DH:(q + 1) * DH])


def _tc2(s_acc, hn, dinv, b):
    return pl.pallas_call(
        _tc2_body,
        out_shape=jax.ShapeDtypeStruct((NP, 2 * D_OUT), jnp.float32),
    )(s_acc, hn, dinv, b)


def kernel(x, edges, W, b):
    e32 = edges.astype(jnp.int32)
    src = e32[:, 0, :]
    dst = e32[:, 1, :]

    pad = EPS - E
    srcp = jnp.pad(src, ((0, 0), (0, pad))).reshape(NSC * NTILES, NCH, CHUNK)
    dstp = jnp.pad(dst, ((0, 0), (0, pad)), constant_values=NP - 1)
    dstp = dstp.reshape(NSC * NTILES, NCH, CHUNK)

    deg_parts = _deg_kernel(dstp.reshape(NSC * NTILES, EPT))

    x_pad = jnp.pad(x, ((0, NP - N), (0, 0)))
    hn, dinv = _tc1(x_pad, W, deg_parts)

    s_acc = _scatter_kernel(hn, srcp, dstp)

    out = _tc2(s_acc, hn, dinv, b.reshape(1, D_OUT))
    return out[:N]


# speedup vs baseline: 1.0005x; 1.0005x over previous
import functools

import jax
import jax.numpy as jnp
from jax import lax
from jax.experimental import pallas as pl
from jax.experimental.pallas import tpu as pltpu
from jax.experimental.pallas import tpu_sc as plsc

N = 10000
E = 320000
D_IN = 128
D_OUT = 64
DH = D_OUT // 2

NP = 10240
NTILES = 16
NSC = 2
CHUNK = 128
NCH = 160
EPT = NCH * CHUNK
EPS = EPT * NTILES
ROWS_PER_TILE = NP // NTILES
NBUF = 8

_MESH = plsc.VectorSubcoreMesh(core_axis_name="c", subcore_axis_name="s")


@functools.partial(
    pl.kernel,
    out_type=jax.ShapeDtypeStruct((NSC * NTILES, NP), jnp.float32),
    mesh=_MESH,
    scratch_types=[
        pltpu.VMEM((EPT,), jnp.int32),
        pltpu.VMEM((NP,), jnp.float32),
    ],
    compiler_params=pltpu.CompilerParams(needs_layout_passes=False),
)
def _deg_kernel(dst_hbm, deg_out, idx_v, deg_v):
    c = lax.axis_index("c").astype(jnp.int32)
    s = lax.axis_index("s").astype(jnp.int32)
    wid = c * jnp.int32(NTILES) + s
    pltpu.sync_copy(dst_hbm.at[wid], idx_v)

    zeros16 = jnp.zeros((16,), jnp.float32)

    @pl.loop(jnp.int32(0), jnp.int32(NP // 16))
    def _zero(i):
        deg_v[pl.ds(pl.multiple_of(i * 16, 16), 16)] = zeros16

    ones16 = jnp.ones((16,), jnp.float32)

    @pl.loop(jnp.int32(0), jnp.int32(EPT // 64))
    def _count(i):
        for j in range(4):
            idx = idx_v[pl.ds(pl.multiple_of(i * 64 + j * 16, 16), 16)]
            plsc.addupdate_scatter(deg_v, [idx], ones16)

    pltpu.sync_copy(deg_v, deg_out.at[wid])


@functools.partial(
    pl.kernel,
    out_type=jax.ShapeDtypeStruct((NSC * 2 * NP, DH), jnp.float32),
    mesh=_MESH,
    scratch_types=[
        pltpu.VMEM((NCH, CHUNK), jnp.int32),
        pltpu.VMEM((NCH, CHUNK), jnp.int32),
        [pltpu.VMEM((CHUNK, DH), jnp.float32) for _ in range(NBUF)],
        pltpu.VMEM_SHARED((NP, DH), jnp.float32),
        pltpu.VMEM_SHARED((NP, DH), jnp.float32),
        [pltpu.SemaphoreType.DMA for _ in range(NBUF)],
        [pltpu.SemaphoreType.DMA for _ in range(NBUF)],
    ],
    compiler_params=pltpu.CompilerParams(use_tc_tiling_on_sc=False),
)
def _scatter_kernel(hn_hbm, src_hbm, dst_hbm, s_out,
                    src_v, dst_v, bufs, s_sh, hn_sh, gsems, ssems):
    c = lax.axis_index("c").astype(jnp.int32)
    s = lax.axis_index("s").astype(jnp.int32)
    wid = c * jnp.int32(NTILES) + s
    tile0 = pl.multiple_of(s * jnp.int32(ROWS_PER_TILE), CHUNK)

    pltpu.sync_copy(src_hbm.at[wid], src_v)
    pltpu.sync_copy(dst_hbm.at[wid], dst_v)

    zeros16 = jnp.zeros((16,), jnp.float32)

    def _gather(ch, b):
        return pltpu.make_async_copy(hn_sh.at[src_v.at[ch]], bufs[b], gsems[b])

    def _scat(ch, b):
        return pltpu.make_async_copy(bufs[b], s_sh.at[dst_v.at[ch]], ssems[b])

    for half in range(2):
        @pl.loop(jnp.int32(0), jnp.int32(CHUNK))
        def _zrow(i):
            for j in range(DH // 16):
                bufs[0][i, pl.ds(j * 16, 16)] = zeros16

        half_base = pl.multiple_of((c * jnp.int32(2) + jnp.int32(half))
                                   * jnp.int32(NP) + tile0, CHUNK)
        pltpu.sync_copy(hn_hbm.at[pl.ds(half_base, ROWS_PER_TILE)],
                        hn_sh.at[pl.ds(tile0, ROWS_PER_TILE)])
        for k in range(ROWS_PER_TILE // CHUNK):
            row0 = pl.multiple_of(tile0 + jnp.int32(k * CHUNK), CHUNK)
            pltpu.sync_copy(bufs[0], s_sh.at[pl.ds(row0, CHUNK)])

        plsc.subcore_barrier()

        for b in range(NBUF):
            _gather(jnp.int32(b), b).start()

        @pl.loop(jnp.int32(0), jnp.int32(NCH), step=jnp.int32(NBUF))
        def _main(g0):
            for b in range(NBUF):
                ch = g0 + b
                _gather(ch, b).wait()
                pltpu.async_copy(bufs[b], s_sh.at[dst_v.at[ch]], ssems[b],
                                 add=True)
                nxt = ch + NBUF

                @pl.when(nxt < NCH)
                def _start_next():
                    _scat(ch, b).wait()
                    _gather(nxt, b).start()

            @pl.when(g0 + jnp.int32(NBUF) >= jnp.int32(NCH))
            def _drain():
                for b in range(NBUF):
                    _scat(g0 + b, b).wait()

        plsc.subcore_barrier()

        pltpu.sync_copy(s_sh.at[pl.ds(tile0, ROWS_PER_TILE)],
                        s_out.at[pl.ds(half_base, ROWS_PER_TILE)])

        if half == 0:
            plsc.subcore_barrier()


def _tc1a_body(x_ref, w_ref, h_ref):
    h_ref[...] = jnp.dot(x_ref[...], w_ref[...],
                         preferred_element_type=jnp.float32)


def _tc1a(x_pad, w):
    return pl.pallas_call(
        _tc1a_body,
        out_shape=jax.ShapeDtypeStruct((NP, D_OUT), jnp.float32),
    )(x_pad, w)


def _tc1b_body(h_ref, deg_ref, hn_ref, dinv_ref):
    h = h_ref[...]
    deg = deg_ref[...].reshape(NSC, NTILES, NP).sum(axis=1) + 1.0
    rows = lax.broadcasted_iota(jnp.int32, (NSC, NP), 1)
    dinv = jnp.where(rows < N, lax.rsqrt(deg), 0.0)
    dinv_ref[...] = dinv
    for e in range(NSC):
        hne = h * dinv[e][:, None]
        for q in range(2):
            base = (e * 2 + q) * NP
            hn_ref[base:base + NP, :] = hne[:, q * DH:(q + 1) * DH]


def _tc1b(h, deg_parts):
    return pl.pallas_call(
        _tc1b_body,
        out_shape=(
            jax.ShapeDtypeStruct((NSC * 2 * NP, DH), jnp.float32),
            jax.ShapeDtypeStruct((NSC, NP), jnp.float32),
        ),
    )(h, deg_parts)


def _tc2_body(s_ref, hn_ref, dinv_ref, b_ref, o_ref):
    dinv = dinv_ref[...]
    bias = b_ref[...]
    for e in range(NSC):
        for q in range(2):
            base = (e * 2 + q) * NP
            col = e * D_OUT + q * DH
            o_ref[:, col:col + DH] = (
                dinv[e][:, None]
                * (s_ref[base:base + NP, :] + hn_ref[base:base + NP, :])
                + bias[:, q * DH:(q + 1) * DH])


def _tc2(s_acc, hn, dinv, b):
    return pl.pallas_call(
        _tc2_body,
        out_shape=jax.ShapeDtypeStruct((NP, 2 * D_OUT), jnp.float32),
    )(s_acc, hn, dinv, b)


def kernel(x, edges, W, b):
    e32 = edges.astype(jnp.int32)
    src = e32[:, 0, :]
    dst = e32[:, 1, :]

    pad = EPS - E
    srcp = jnp.pad(src, ((0, 0), (0, pad))).reshape(NSC * NTILES, NCH, CHUNK)
    dstp = jnp.pad(dst, ((0, 0), (0, pad)), constant_values=NP - 1)
    dstp = dstp.reshape(NSC * NTILES, NCH, CHUNK)

    x_pad = jnp.pad(x, ((0, NP - N), (0, 0)))
    h = _tc1a(x_pad, W)
    deg_parts = _deg_kernel(dstp.reshape(NSC * NTILES, EPT))
    hn, dinv = _tc1b(h, deg_parts)

    s_acc = _scatter_kernel(hn, srcp, dstp)

    out = _tc2(s_acc, hn, dinv, b.reshape(1, D_OUT))
    return out[:N]
